# baseline (speedup 1.0000x reference)
import jax
import jax.numpy as jnp
from jax.experimental import pallas as pl
from jax.experimental.pallas import tpu as pltpu

NUM_EXPERTS = 64
TOP_K = 8
BT = 1024


def _router_kernel(x_ref, wt_ref, iota_ref, logits_ref, idx_ref, val_ref):
    iota_col = iota_ref[...]
    logits = jnp.dot(x_ref[...], wt_ref[...], preferred_element_type=jnp.float32)
    logits_ref[...] = logits

    work = logits.T
    vals = []
    idxs = []
    for _ in range(TOP_K):
        m = jnp.max(work, axis=0, keepdims=True)
        at = work == m
        idxs.append(jnp.sum(jnp.where(at, iota_col, 0.0), axis=0, keepdims=True))
        vals.append(m)
        work = jnp.where(at, -jnp.inf, work)
    v = jnp.concatenate(vals, axis=0)
    idxf = jnp.concatenate(idxs, axis=0)
    idx_ref[...] = idxf.T.astype(jnp.int32)

    p = jnp.exp(v - v[:1, :])
    val_ref[...] = (p / jnp.sum(p, axis=0, keepdims=True)).T


@jax.jit
def _router(x_flat, wt, iota_col):
    t = x_flat.shape[0]
    grid = (t // BT,)
    return pl.pallas_call(
        _router_kernel,
        grid=grid,
        in_specs=[
            pl.BlockSpec((BT, x_flat.shape[1]), lambda i: (i, 0)),
            pl.BlockSpec((wt.shape[0], NUM_EXPERTS), lambda i: (0, 0)),
            pl.BlockSpec((NUM_EXPERTS, 1), lambda i: (0, 0)),
        ],
        out_specs=[
            pl.BlockSpec((BT, NUM_EXPERTS), lambda i: (i, 0)),
            pl.BlockSpec((BT, TOP_K), lambda i: (i, 0)),
            pl.BlockSpec((BT, TOP_K), lambda i: (i, 0)),
        ],
        out_shape=[
            jax.ShapeDtypeStruct((t, NUM_EXPERTS), jnp.float32),
            jax.ShapeDtypeStruct((t, TOP_K), jnp.int32),
            jax.ShapeDtypeStruct((t, TOP_K), jnp.float32),
        ],
        compiler_params=pltpu.CompilerParams(
            dimension_semantics=("parallel",),
        ),
    )(x_flat, wt, iota_col)


def kernel(hidden_states, weight, top_k):
    batch_size, seq_len, hidden_size = hidden_states.shape
    x_flat = hidden_states.reshape(-1, hidden_size).astype(jnp.float32)
    wt = weight.astype(jnp.float32).T
    num_exp = weight.shape[0]
    iota_col = jnp.arange(num_exp, dtype=jnp.float32).reshape(num_exp, 1)
    logits, idx, vals = _router(x_flat, wt, iota_col)
    logits = logits.reshape(batch_size, seq_len, num_exp)
    idx = idx.reshape(batch_size, seq_len, TOP_K)
    idx = idx + (jnp.asarray(top_k) - TOP_K).astype(idx.dtype)
    vals = vals.reshape(batch_size, seq_len, TOP_K)
    return (logits, idx, vals)


# speedup vs baseline: 1.0040x; 1.0040x over previous
import jax
import jax.numpy as jnp
from jax.experimental import pallas as pl
from jax.experimental.pallas import tpu as pltpu

NUM_EXPERTS = 64
TOP_K = 8
BT = 2048


def _router_kernel(x_ref, wt_ref, iota_ref, logits_ref, idx_ref, val_ref):
    iota_col = iota_ref[...]
    logits = jnp.dot(x_ref[...], wt_ref[...], preferred_element_type=jnp.float32)
    logits_ref[...] = logits

    work = logits.T
    vals = []
    idxs = []
    for _ in range(TOP_K):
        m = jnp.max(work, axis=0, keepdims=True)
        at = work == m
        idxs.append(jnp.sum(jnp.where(at, iota_col, 0.0), axis=0, keepdims=True))
        vals.append(m)
        work = jnp.where(at, -jnp.inf, work)
    v = jnp.concatenate(vals, axis=0)
    idxf = jnp.concatenate(idxs, axis=0)
    idx_ref[...] = idxf.T.astype(jnp.int32)

    p = jnp.exp(v - v[:1, :])
    val_ref[...] = (p / jnp.sum(p, axis=0, keepdims=True)).T


@jax.jit
def _router(x_flat, wt, iota_col):
    t = x_flat.shape[0]
    grid = (t // BT,)
    return pl.pallas_call(
        _router_kernel,
        grid=grid,
        in_specs=[
            pl.BlockSpec((BT, x_flat.shape[1]), lambda i: (i, 0)),
            pl.BlockSpec((wt.shape[0], NUM_EXPERTS), lambda i: (0, 0)),
            pl.BlockSpec((NUM_EXPERTS, 1), lambda i: (0, 0)),
        ],
        out_specs=[
            pl.BlockSpec((BT, NUM_EXPERTS), lambda i: (i, 0)),
            pl.BlockSpec((BT, TOP_K), lambda i: (i, 0)),
            pl.BlockSpec((BT, TOP_K), lambda i: (i, 0)),
        ],
        out_shape=[
            jax.ShapeDtypeStruct((t, NUM_EXPERTS), jnp.float32),
            jax.ShapeDtypeStruct((t, TOP_K), jnp.int32),
            jax.ShapeDtypeStruct((t, TOP_K), jnp.float32),
        ],
        compiler_params=pltpu.CompilerParams(
            dimension_semantics=("parallel",),
        ),
    )(x_flat, wt, iota_col)


def kernel(hidden_states, weight, top_k):
    batch_size, seq_len, hidden_size = hidden_states.shape
    x_flat = hidden_states.reshape(-1, hidden_size).astype(jnp.float32)
    wt = weight.astype(jnp.float32).T
    num_exp = weight.shape[0]
    iota_col = jnp.arange(num_exp, dtype=jnp.float32).reshape(num_exp, 1)
    logits, idx, vals = _router(x_flat, wt, iota_col)
    logits = logits.reshape(batch_size, seq_len, num_exp)
    idx = idx.reshape(batch_size, seq_len, TOP_K)
    idx = idx + (jnp.asarray(top_k) - TOP_K).astype(idx.dtype)
    vals = vals.reshape(batch_size, seq_len, TOP_K)
    return (logits, idx, vals)
